# baseline (speedup 1.0000x reference)
import jax
import jax.numpy as jnp
from jax.experimental import pallas as pl
from jax.experimental.pallas import tpu as pltpu

def _make_body(Wc, Wo, tr, ns):
    def _body(*refs):
        x_refs = refs[:ns]
        sel_ref = refs[ns]
        o_ref = refs[ns + 1]
        sel = sel_ref[...]
        for k, x_ref in enumerate(x_refs):
            xb = x_ref[...]
            rows = xb[:, :Wc] + xb[:, Wc:]
            hi = rows.astype(jnp.bfloat16)
            lo = (rows - hi.astype(jnp.float32)).astype(jnp.bfloat16)
            acc = jnp.dot(hi, sel, preferred_element_type=jnp.float32)
            acc += jnp.dot(lo, sel, preferred_element_type=jnp.float32)
            o_ref[k * tr:(k + 1) * tr, :] = acc.astype(o_ref.dtype)

    return _body


@jax.jit
def _avg_pool_2x2(x):
    N, C, H, W = x.shape
    Ho, Wo = H // 2, W // 2
    if Ho == 0 or Wo == 0:
        return jnp.zeros((N, C, Ho, Wo), x.dtype)
    Wc = 2 * Wo
    xc = x[:, :, : 2 * Ho, :Wc]

    R = N * C * Ho
    x2 = xc.reshape(R, 2 * Wc)

    ii = jax.lax.broadcasted_iota(jnp.int32, (Wc, Wo), 0)
    jj = jax.lax.broadcasted_iota(jnp.int32, (Wc, Wo), 1)
    sel = jnp.where(ii // 2 == jj, 0.25, 0.0).astype(jnp.bfloat16)

    ns, tr = 1, R
    for cand_ns in (4, 2, 1):
        cands = [c for c in (1024, 512, 256, 128, 64, 32, 16, 8, 4, 2, 1)
                 if R % (c * cand_ns) == 0]
        if cands:
            ns, tr = cand_ns, cands[0]
            break
    grid = (R // (tr * ns),)

    def _in_spec(k):
        return pl.BlockSpec((tr, 2 * Wc), lambda r, _k=k: (ns * r + _k, 0))

    out2 = pl.pallas_call(
        _make_body(Wc, Wo, tr, ns),
        out_shape=jax.ShapeDtypeStruct((R, Wo), x.dtype),
        grid=grid,
        in_specs=[_in_spec(k) for k in range(ns)]
        + [pl.BlockSpec((Wc, Wo), lambda r: (0, 0))],
        out_specs=pl.BlockSpec((tr * ns, Wo), lambda r: (r, 0)),
        compiler_params=pltpu.CompilerParams(
            dimension_semantics=("arbitrary",),
            vmem_limit_bytes=100 * 1024 * 1024,
        ),
    )(*([x2] * ns), sel)

    return out2.reshape(N, C, Ho, Wo)


def kernel(x):
    return _avg_pool_2x2(x)
